# baseline (speedup 1.0000x reference)
import functools

import jax
import jax.numpy as jnp
from jax import lax
from jax.experimental import pallas as pl
from jax.experimental.pallas import tpu as pltpu
from jax.experimental.pallas import tpu_sc as plsc

B, S, DIM = 4, 8192, 128
N = B * S

_info = plsc.get_sparse_core_info()
NC, NS, L = _info.num_cores, _info.num_subcores, _info.num_lanes
NW = NC * NS
C = N // NW
K = 64
NSUB = C // K
NBUF = 2
DUMP = N

_mesh = plsc.VectorSubcoreMesh(core_axis_name="c", subcore_axis_name="s")


@functools.partial(
    pl.kernel,
    mesh=_mesh,
    out_type=jax.ShapeDtypeStruct((N + 8, DIM), jnp.float32),
    scratch_types=(
        [pltpu.VMEM((C,), jnp.int32), pltpu.VMEM((C,), jnp.int32)]
        + [pltpu.VMEM((K, DIM), jnp.float32) for _ in range(4 * NBUF)]
        + [pltpu.VMEM((K,), jnp.int32) for _ in range(4 * NBUF)]
        + [pltpu.SemaphoreType.DMA for _ in range(2 * NBUF)]
    ),
)
def _sc_lookup(ids_hbm, mods_hbm, t0, t1, t2, t3, out_hbm, *scratch):
    ids_v, mods_v = scratch[0], scratch[1]
    bufs = scratch[2 : 2 + 4 * NBUF]
    poss = scratch[2 + 4 * NBUF : 2 + 8 * NBUF]
    gsems = scratch[2 + 8 * NBUF : 2 + 8 * NBUF + NBUF]
    ssems = scratch[2 + 8 * NBUF + NBUF :]
    tables = (t0, t1, t2, t3)

    wid = lax.axis_index("s") * NC + lax.axis_index("c")
    base = wid * C
    pltpu.sync_copy(ids_hbm.at[pl.ds(base, C)], ids_v)
    pltpu.sync_copy(mods_hbm.at[pl.ds(base, C)], mods_v)

    gather_d = [None] * NBUF
    scatter_d = [None] * NBUF

    for s in range(NSUB):
        u = s % NBUF
        off = s * K
        if scatter_d[u] is not None:
            for d in scatter_d[u]:
                d.wait()
        gather_d[u] = [
            pltpu.async_copy(
                tables[t].at[ids_v.at[pl.ds(off, K)]], bufs[4 * u + t], gsems[u]
            )
            for t in range(4)
        ]
        for g in range(K // L):
            sl = pl.ds(off + g * L, L)
            mv = mods_v[sl]
            posv = jnp.arange(L, dtype=jnp.int32) + (base + off + g * L)
            for t in range(4):
                poss[4 * u + t][pl.ds(g * L, L)] = jnp.where(
                    mv == t, posv, jnp.int32(DUMP)
                )
        for d in gather_d[u]:
            d.wait()
        scatter_d[u] = [
            pltpu.async_copy(
                bufs[4 * u + t], out_hbm.at[poss[4 * u + t]], ssems[u]
            )
            for t in range(4)
        ]
    for u in range(NBUF):
        if scatter_d[u] is not None:
            for d in scatter_d[u]:
                d.wait()


def kernel(input_ids, modality_ids, text_table, image_table, video_table, audio_table):
    ids = input_ids.reshape(-1)
    mods = modality_ids.reshape(-1)
    out = _sc_lookup(ids, mods, text_table, image_table, video_table, audio_table)
    return out[:N].reshape(B, S, DIM)


# speedup vs baseline: 46.4502x; 46.4502x over previous
import functools

import jax
import jax.numpy as jnp
from jax import lax
from jax.experimental import pallas as pl
from jax.experimental.pallas import tpu as pltpu
from jax.experimental.pallas import tpu_sc as plsc

B, S, DIM = 4, 8192, 128
N = B * S

_info = plsc.get_sparse_core_info()
NC, NS, L = _info.num_cores, _info.num_subcores, _info.num_lanes
NW = NC * NS
C = N // NW
K = 64
NSUB = C // K
NBUF = 2
NDUMP = NW * 4 * K

_mesh = plsc.VectorSubcoreMesh(core_axis_name="c", subcore_axis_name="s")


@functools.partial(
    pl.kernel,
    mesh=_mesh,
    out_type=jax.ShapeDtypeStruct((N + NDUMP, DIM), jnp.float32),
    scratch_types=(
        [pltpu.VMEM((C,), jnp.int32), pltpu.VMEM((C,), jnp.int32)]
        + [pltpu.VMEM((K, DIM), jnp.float32) for _ in range(4 * NBUF)]
        + [pltpu.VMEM((K,), jnp.int32) for _ in range(4 * NBUF)]
        + [pltpu.SemaphoreType.DMA for _ in range(2 * NBUF)]
    ),
)
def _sc_lookup(ids_hbm, mods_hbm, t0, t1, t2, t3, out_hbm, *scratch):
    ids_v, mods_v = scratch[0], scratch[1]
    bufs = scratch[2 : 2 + 4 * NBUF]
    poss = scratch[2 + 4 * NBUF : 2 + 8 * NBUF]
    gsems = scratch[2 + 8 * NBUF : 2 + 8 * NBUF + NBUF]
    ssems = scratch[2 + 8 * NBUF + NBUF :]
    tables = (t0, t1, t2, t3)

    wid = lax.axis_index("s") * NC + lax.axis_index("c")
    base = wid * C
    pltpu.sync_copy(ids_hbm.at[pl.ds(base, C)], ids_v)
    pltpu.sync_copy(mods_hbm.at[pl.ds(base, C)], mods_v)

    gather_d = [None] * NBUF
    scatter_d = [None] * NBUF

    for s in range(NSUB):
        u = s % NBUF
        off = s * K
        if scatter_d[u] is not None:
            for d in scatter_d[u]:
                d.wait()
        gather_d[u] = [
            pltpu.async_copy(
                tables[t].at[ids_v.at[pl.ds(off, K)]], bufs[4 * u + t], gsems[u]
            )
            for t in range(4)
        ]
        for g in range(K // L):
            sl = pl.ds(off + g * L, L)
            mv = mods_v[sl]
            posv = jnp.arange(L, dtype=jnp.int32) + (base + off + g * L)
            dump_base = N + wid * (4 * K) + g * L
            for t in range(4):
                dumpv = jnp.arange(L, dtype=jnp.int32) + (dump_base + t * K)
                poss[4 * u + t][pl.ds(g * L, L)] = jnp.where(mv == t, posv, dumpv)
        for d in gather_d[u]:
            d.wait()
        scatter_d[u] = [
            pltpu.async_copy(
                bufs[4 * u + t], out_hbm.at[poss[4 * u + t]], ssems[u]
            )
            for t in range(4)
        ]
    for u in range(NBUF):
        if scatter_d[u] is not None:
            for d in scatter_d[u]:
                d.wait()


def kernel(input_ids, modality_ids, text_table, image_table, video_table, audio_table):
    ids = input_ids.reshape(-1)
    mods = modality_ids.reshape(-1)
    out = _sc_lookup(ids, mods, text_table, image_table, video_table, audio_table)
    return out[:N].reshape(B, S, DIM)
